# baseline (speedup 1.0000x reference)
import functools
import math

import jax
import jax.numpy as jnp
from jax import lax
from jax.experimental import pallas as pl
from jax.experimental.pallas import tpu as pltpu
from jax.experimental.pallas import tpu_sc as plsc

SEQ = 8192
D = 512
VOCAB = 202

NC = 2
NS = 16
NW = NC * NS
ROWS_PER_W = SEQ // NW
CHUNK = 64
NCHUNK = ROWS_PER_W // CHUNK
NG = SEQ // CHUNK
NVC = D // 16

_NEG2LOG1E4_D = -2.0 * math.log(10000.0) / D


def _inv_denom(shape):
    c = lax.broadcasted_iota(jnp.int32, shape, 1)
    return jnp.exp((c >> 1).astype(jnp.float32) * _NEG2LOG1E4_D)


def _basis_body(cl_ref, sl_ref, u_ref, v_ref):
    inv = _inv_denom((CHUNK, D))
    l = lax.broadcasted_iota(jnp.int32, (CHUNK, D), 0).astype(jnp.float32)
    sl_ref[...] = jnp.sin(l * inv)
    cl_ref[...] = jnp.cos(l * inv)

    inv1 = _inv_denom((16, D))
    q = lax.broadcasted_iota(jnp.int32, (16, D), 0).astype(jnp.float32)
    a1 = q * 512.0 * inv1
    s1 = jnp.sin(a1)
    c1 = jnp.cos(a1)
    inv2 = _inv_denom((8, D))
    p = lax.broadcasted_iota(jnp.int32, (8, D), 0).astype(jnp.float32)
    a2 = p * 64.0 * inv2
    s2 = jnp.sin(a2)
    c2 = jnp.cos(a2)
    cc = lax.broadcasted_iota(jnp.int32, (8, D), 1)
    even = (cc & 1) == 0
    live = cc < D - 2
    for qi in range(16):
        sh = s1[qi : qi + 1, :] * c2 + c1[qi : qi + 1, :] * s2
        ch = c1[qi : qi + 1, :] * c2 - s1[qi : qi + 1, :] * s2
        u_ref[qi * 8 : (qi + 1) * 8, :] = jnp.where(
            even & live, sh, jnp.where(live, ch, 0.0)
        )
        v_ref[qi * 8 : (qi + 1) * 8, :] = jnp.where(
            even & live, ch, jnp.where(live, -sh, 0.0)
        )


def _basis():
    shp = jax.ShapeDtypeStruct
    return pl.pallas_call(
        _basis_body,
        out_shape=(
            shp((CHUNK, D), jnp.float32),
            shp((CHUNK, D), jnp.float32),
            shp((NG, D), jnp.float32),
            shp((NG, D), jnp.float32),
        ),
    )()


def _sc_body(
    x_hbm,
    tab_hbm,
    cl_hbm,
    sl_hbm,
    u_hbm,
    v_hbm,
    o_hbm,
    idx_v,
    rows_v,
    cl_v,
    sl_v,
    u_v,
    v_v,
    sem,
):
    wid = lax.axis_index("s") * NC + lax.axis_index("c")
    base = wid * ROWS_PER_W

    pltpu.sync_copy(x_hbm.at[pl.ds(base, ROWS_PER_W)], idx_v)
    pltpu.sync_copy(cl_hbm, cl_v)
    pltpu.sync_copy(sl_hbm, sl_v)
    g0 = wid * NCHUNK
    pltpu.sync_copy(u_hbm.at[pl.ds(g0, NCHUNK)], u_v)
    pltpu.sync_copy(v_hbm.at[pl.ds(g0, NCHUNK)], v_v)

    for h in range(NCHUNK):
        pltpu.async_copy(
            tab_hbm.at[idx_v.at[pl.ds(h * CHUNK, CHUNK)]], rows_v, sem
        ).wait()

        def _row(l, _):
            for vc in range(NVC):
                s = pl.ds(vc * 16, 16)
                pe = u_v[h, s] * cl_v[l, s] + v_v[h, s] * sl_v[l, s]
                rows_v[l, s] = rows_v[l, s] + pe
            return _

        lax.fori_loop(0, CHUNK, _row, 0)
        pltpu.sync_copy(rows_v, o_hbm.at[pl.ds(base + h * CHUNK, CHUNK)])


def _sc_call(x, wordlist, cl, sl, u, v):
    mesh = plsc.VectorSubcoreMesh(
        core_axis_name="c", subcore_axis_name="s", num_cores=NC, num_subcores=NS
    )
    f = pl.kernel(
        _sc_body,
        out_type=jax.ShapeDtypeStruct((SEQ, D), jnp.float32),
        mesh=mesh,
        scratch_types=[
            pltpu.VMEM((ROWS_PER_W,), jnp.int32),
            pltpu.VMEM((CHUNK, D), jnp.float32),
            pltpu.VMEM((CHUNK, D), jnp.float32),
            pltpu.VMEM((CHUNK, D), jnp.float32),
            pltpu.VMEM((NCHUNK, D), jnp.float32),
            pltpu.VMEM((NCHUNK, D), jnp.float32),
            pltpu.SemaphoreType.DMA,
        ],
    )
    return f(x, wordlist, cl, sl, u, v)


@functools.partial(jax.jit)
def kernel(x, wordlist):
    cl, sl, u, v = _basis()
    return _sc_call(x, wordlist, cl, sl, u, v)


# speedup vs baseline: 1.7501x; 1.7501x over previous
import functools
import math

import jax
import jax.numpy as jnp
from jax import lax
from jax.experimental import pallas as pl
from jax.experimental.pallas import tpu as pltpu
from jax.experimental.pallas import tpu_sc as plsc

SEQ = 8192
D = 512
VOCAB = 202

NC = 2
NS = 16
NW = NC * NS
ROWS_PER_W = SEQ // NW
CHUNK = 32
NCHUNK = ROWS_PER_W // CHUNK
NG = SEQ // 64
NVC = D // 16

_NEG2LOG1E4_D = -2.0 * math.log(10000.0) / D


def _inv_denom(shape):
    c = lax.broadcasted_iota(jnp.int32, shape, 1)
    return jnp.exp((c >> 1).astype(jnp.float32) * _NEG2LOG1E4_D)


def _basis_body(cl_ref, sl_ref, u_ref, v_ref):
    inv = _inv_denom((64, D))
    l = lax.broadcasted_iota(jnp.int32, (64, D), 0).astype(jnp.float32)
    sl_ref[...] = jnp.sin(l * inv)
    cl_ref[...] = jnp.cos(l * inv)

    inv1 = _inv_denom((16, D))
    q = lax.broadcasted_iota(jnp.int32, (16, D), 0).astype(jnp.float32)
    a1 = q * 512.0 * inv1
    s1 = jnp.sin(a1)
    c1 = jnp.cos(a1)
    inv2 = _inv_denom((8, D))
    p = lax.broadcasted_iota(jnp.int32, (8, D), 0).astype(jnp.float32)
    a2 = p * 64.0 * inv2
    s2 = jnp.sin(a2)
    c2 = jnp.cos(a2)
    cc = lax.broadcasted_iota(jnp.int32, (8, D), 1)
    even = (cc & 1) == 0
    live = cc < D - 2
    for qi in range(16):
        sh = s1[qi : qi + 1, :] * c2 + c1[qi : qi + 1, :] * s2
        ch = c1[qi : qi + 1, :] * c2 - s1[qi : qi + 1, :] * s2
        u_ref[qi * 8 : (qi + 1) * 8, :] = jnp.where(
            even & live, sh, jnp.where(live, ch, 0.0)
        )
        v_ref[qi * 8 : (qi + 1) * 8, :] = jnp.where(
            even & live, ch, jnp.where(live, -sh, 0.0)
        )


def _basis():
    shp = jax.ShapeDtypeStruct
    return pl.pallas_call(
        _basis_body,
        out_shape=(
            shp((64, D), jnp.float32),
            shp((64, D), jnp.float32),
            shp((NG, D), jnp.float32),
            shp((NG, D), jnp.float32),
        ),
    )()


def _sc_body(
    x_hbm,
    tab_hbm,
    cl_hbm,
    sl_hbm,
    u_hbm,
    v_hbm,
    o_hbm,
    idx_v,
    rows0,
    rows1,
    cl_v,
    sl_v,
    u_v,
    v_v,
    gs0,
    gs1,
    ws0,
    ws1,
):
    wid = lax.axis_index("s") * NC + lax.axis_index("c")
    base = wid * ROWS_PER_W

    pltpu.sync_copy(x_hbm.at[pl.ds(base, ROWS_PER_W)], idx_v)
    pltpu.sync_copy(cl_hbm, cl_v)
    pltpu.sync_copy(sl_hbm, sl_v)
    g0 = wid * (ROWS_PER_W // 64)
    pltpu.sync_copy(u_hbm.at[pl.ds(g0, ROWS_PER_W // 64)], u_v)
    pltpu.sync_copy(v_hbm.at[pl.ds(g0, ROWS_PER_W // 64)], v_v)

    bufs = (rows0, rows1)
    gsems = (gs0, gs1)
    wsems = (ws0, ws1)

    def _fire_gather(h):
        return pltpu.async_copy(
            tab_hbm.at[idx_v.at[pl.ds(h * CHUNK, CHUNK)]],
            bufs[h & 1],
            gsems[h & 1],
        )

    def _compute(buf, h, vg):
        uu = [u_v[h // 2, pl.ds((vg * 8 + j) * 16, 16)] for j in range(8)]
        vv = [v_v[h // 2, pl.ds((vg * 8 + j) * 16, 16)] for j in range(8)]
        lo = (h & 1) * CHUNK

        def _row(l, carry):
            for j in range(8):
                s = pl.ds((vg * 8 + j) * 16, 16)
                pe = uu[j] * cl_v[lo + l, s] + vv[j] * sl_v[lo + l, s]
                buf[l, s] = buf[l, s] + pe
            return carry

        lax.fori_loop(0, CHUNK, _row, 0)

    gh = [None] * NCHUNK
    wh = [None] * NCHUNK
    gh[0] = _fire_gather(0)
    for h in range(NCHUNK):
        b = h & 1
        gh[h].wait()
        _compute(bufs[b], h, 0)
        _compute(bufs[b], h, 1)
        if h + 1 < NCHUNK:
            if h >= 1:
                wh[h - 1].wait()
            gh[h + 1] = _fire_gather(h + 1)
        _compute(bufs[b], h, 2)
        _compute(bufs[b], h, 3)
        wh[h] = pltpu.async_copy(
            bufs[b], o_hbm.at[pl.ds(base + h * CHUNK, CHUNK)], wsems[b]
        )
    wh[NCHUNK - 2].wait()
    wh[NCHUNK - 1].wait()


def _sc_call(x, wordlist, cl, sl, u, v):
    mesh = plsc.VectorSubcoreMesh(
        core_axis_name="c", subcore_axis_name="s", num_cores=NC, num_subcores=NS
    )
    f = pl.kernel(
        _sc_body,
        out_type=jax.ShapeDtypeStruct((SEQ, D), jnp.float32),
        mesh=mesh,
        scratch_types=[
            pltpu.VMEM((ROWS_PER_W,), jnp.int32),
            pltpu.VMEM((CHUNK, D), jnp.float32),
            pltpu.VMEM((CHUNK, D), jnp.float32),
            pltpu.VMEM((64, D), jnp.float32),
            pltpu.VMEM((64, D), jnp.float32),
            pltpu.VMEM((ROWS_PER_W // 64, D), jnp.float32),
            pltpu.VMEM((ROWS_PER_W // 64, D), jnp.float32),
            pltpu.SemaphoreType.DMA,
            pltpu.SemaphoreType.DMA,
            pltpu.SemaphoreType.DMA,
            pltpu.SemaphoreType.DMA,
        ],
    )
    return f(x, wordlist, cl, sl, u, v)


@functools.partial(jax.jit)
def kernel(x, wordlist):
    cl, sl, u, v = _basis()
    return _sc_call(x, wordlist, cl, sl, u, v)


# speedup vs baseline: 8.5358x; 4.8773x over previous
import functools
import math

import jax
import jax.numpy as jnp
from jax import lax
from jax.experimental import pallas as pl
from jax.experimental.pallas import tpu as pltpu

SEQ = 8192
D = 512
VOCAB = 202
VPAD = 256
BLK = 2048
GRID = SEQ // BLK
NH = BLK // 64
NG = SEQ // 64

_NEG2LOG1E4_D = -2.0 * math.log(10000.0) / D


def _inv_denom(shape):
    c = lax.broadcasted_iota(jnp.int32, shape, 1)
    return jnp.exp((c >> 1).astype(jnp.float32) * _NEG2LOG1E4_D)


def _body(x_ref, w_ref, o_ref, sl_ref, cl_ref, u_ref, v_ref, hi_ref, lo_ref):
    b = pl.program_id(0)

    @pl.when(b == 0)
    def _init():
        w = w_ref[...]
        hi = w.astype(jnp.bfloat16)
        hi_ref[...] = hi
        lo_ref[...] = (w - hi.astype(jnp.float32)).astype(jnp.bfloat16)

        inv = _inv_denom((64, D))
        l = lax.broadcasted_iota(jnp.int32, (64, D), 0).astype(jnp.float32)
        sl_ref[...] = jnp.sin(l * inv)
        cl_ref[...] = jnp.cos(l * inv)

        inv1 = _inv_denom((16, D))
        q = lax.broadcasted_iota(jnp.int32, (16, D), 0).astype(jnp.float32)
        a1 = q * 512.0 * inv1
        s1 = jnp.sin(a1)
        c1 = jnp.cos(a1)
        inv2 = _inv_denom((8, D))
        p = lax.broadcasted_iota(jnp.int32, (8, D), 0).astype(jnp.float32)
        a2 = p * 64.0 * inv2
        s2 = jnp.sin(a2)
        c2 = jnp.cos(a2)
        cc = lax.broadcasted_iota(jnp.int32, (8, D), 1)
        even = (cc & 1) == 0
        live = cc < D - 2
        for qi in range(16):
            sh = s1[qi : qi + 1, :] * c2 + c1[qi : qi + 1, :] * s2
            ch = c1[qi : qi + 1, :] * c2 - s1[qi : qi + 1, :] * s2
            u_ref[qi * 8 : (qi + 1) * 8, :] = jnp.where(
                even & live, sh, jnp.where(live, ch, 0.0)
            )
            v_ref[qi * 8 : (qi + 1) * 8, :] = jnp.where(
                even & live, ch, jnp.where(live, -sh, 0.0)
            )

    idx = x_ref[0, 0, :]
    votes = lax.broadcasted_iota(jnp.int32, (BLK, VPAD), 1)
    onehot = (idx[:, None] == votes).astype(jnp.bfloat16)
    g = jnp.dot(onehot, hi_ref[...], preferred_element_type=jnp.float32)
    g = g + jnp.dot(onehot, lo_ref[...], preferred_element_type=jnp.float32)

    us = u_ref[pl.ds(b * NH, NH), :]
    vs = v_ref[pl.ds(b * NH, NH), :]
    cl = cl_ref[...]
    sl = sl_ref[...]
    for h in range(NH):
        pm = us[h : h + 1, :] * cl + vs[h : h + 1, :] * sl
        o_ref[h * 64 : (h + 1) * 64, :] = g[h * 64 : (h + 1) * 64, :] + pm


@functools.partial(jax.jit)
def kernel(x, wordlist):
    xb = x.reshape(GRID, 1, BLK)
    wp = jnp.pad(wordlist, ((0, VPAD - VOCAB), (0, 0)))
    return pl.pallas_call(
        _body,
        grid=(GRID,),
        in_specs=[
            pl.BlockSpec((1, 1, BLK), lambda b: (b, 0, 0)),
            pl.BlockSpec((VPAD, D), lambda b: (0, 0)),
        ],
        out_specs=pl.BlockSpec((BLK, D), lambda b: (b, 0)),
        out_shape=jax.ShapeDtypeStruct((SEQ, D), jnp.float32),
        scratch_shapes=[
            pltpu.VMEM((64, D), jnp.float32),
            pltpu.VMEM((64, D), jnp.float32),
            pltpu.VMEM((NG, D), jnp.float32),
            pltpu.VMEM((NG, D), jnp.float32),
            pltpu.VMEM((VPAD, D), jnp.bfloat16),
            pltpu.VMEM((VPAD, D), jnp.bfloat16),
        ],
    )(xb, wp)
